# baseline (speedup 1.0000x reference)
import jax
import jax.numpy as jnp
from jax import lax
from jax.experimental import pallas as pl
from jax.experimental.pallas import tpu as pltpu
from jax.experimental.pallas import tpu_sc as plsc

N = 10000
NP = 10240
E = 320000
NC = 2
NS = 16
NW = NC * NS
C = 128
G = 84
EP = NW * G * C
K = 1
NBLK = G // K
RPT = NP // NS
D = 128
DH = 64
H = 128
OUT = 64
BR = 512
NB = NP // BR

_f32 = jnp.float32


def _make_agg_body(nh):
    def body(u, src3, dst3, zrows, out, sidx, didx, gbuf, acc, us, *sems):
        semg = sems[:3]
        semw = sems[3:]
        c = lax.axis_index("c")
        s = lax.axis_index("s")
        wid = c * NS + s
        rsl = pl.ds(s * RPT, RPT)
        pltpu.sync_copy(src3.at[wid], sidx)
        pltpu.sync_copy(dst3.at[wid], didx)

        for h in range(nh):
            pltpu.sync_copy(u.at[h].at[rsl], us.at[rsl])
            uh = us

            def gfire(blk, set_i):
                for j in range(K):
                    pltpu.async_copy(uh.at[sidx.at[K * blk + j]],
                                     gbuf.at[K * set_i + j], semg[set_i])

            def gdrain(set_i):
                for j in range(K):
                    pltpu.make_async_copy(uh.at[sidx.at[0]],
                                          gbuf.at[K * set_i + j],
                                          semg[set_i]).wait()

            def sfire(blk, set_i):
                for j in range(K):
                    pltpu.async_copy(gbuf.at[K * set_i + j],
                                     acc.at[didx.at[K * blk + j]],
                                     semw[set_i], add=True)

            def sdrain(set_i):
                for j in range(K):
                    pltpu.make_async_copy(gbuf.at[K * set_i + j],
                                          acc.at[didx.at[0]],
                                          semw[set_i]).wait()

            pltpu.sync_copy(zrows.at[rsl], acc.at[rsl])
            plsc.subcore_barrier()

            gfire(0, 0)

            def loop(i, carry):
                for pos in range(3):
                    set_nxt = (pos + 1) % 3
                    blk = 3 * i + pos

                    if pos == 2:
                        sdrain(set_nxt)
                    else:
                        @pl.when(i >= 1)
                        def _():
                            sdrain(set_nxt)

                    if pos == 2:
                        @pl.when(i < NBLK // 3 - 1)
                        def _():
                            gfire(blk + 1, set_nxt)
                    else:
                        gfire(blk + 1, set_nxt)

                    gdrain(pos)
                    sfire(blk, pos)
                return carry

            lax.fori_loop(0, NBLK // 3, loop, 0)
            sdrain(1)
            sdrain(2)
            plsc.subcore_barrier()
            pltpu.sync_copy(acc.at[rsl], out.at[c].at[h].at[rsl])
            if h + 1 < nh:
                plsc.subcore_barrier()
    return body


_sc_mesh = plsc.VectorSubcoreMesh(core_axis_name="c", subcore_axis_name="s")


def _make_agg_kernel(nh):
    return pl.kernel(
        _make_agg_body(nh),
        out_type=jax.ShapeDtypeStruct((NC, nh, NP, DH), _f32),
        mesh=_sc_mesh,
        scratch_types=[
            pltpu.VMEM((G, C), jnp.int32),
            pltpu.VMEM((G, C), jnp.int32),
            pltpu.VMEM((3 * K, C, DH), _f32),
            pltpu.VMEM_SHARED((NP, DH), _f32),
            pltpu.VMEM_SHARED((NP, DH), _f32),
        ] + [pltpu.SemaphoreType.DMA] * 6,
        compiler_params=pltpu.CompilerParams(use_tc_tiling_on_sc=False),
    )


_agg2_kernel = _make_agg_kernel(2)
_agg1_kernel = _make_agg_kernel(1)


def _dinv_body(dg, o):
    i = pl.program_id(0)
    deg = dg[0, 0][:, 0:1] + dg[1, 0][:, 0:1] + 1.0
    r = lax.rsqrt(deg)
    rows = i * BR + lax.broadcasted_iota(jnp.int32, (BR, 1), 0)
    r = jnp.where(rows < N, r, 0.0)
    o[...] = jnp.broadcast_to(r, (BR, D))


def _emb_body(x, we, be, w1, dinv, o):
    h0 = jnp.dot(x[...], we[...], preferred_element_type=_f32) + be[...]
    u = jnp.dot(h0, w1[...], preferred_element_type=_f32) * dinv[...]
    o[0] = u[:, :DH]
    o[1] = u[:, DH:]


def _gather_h(p, u, dinv, b):
    pL = p[0, 0] + p[1, 0] + u[0]
    pR = p[0, 1] + p[1, 1] + u[1]
    agg = jnp.concatenate([pL, pR], axis=1)
    return jnp.maximum(agg * dinv[...] + b[...], 0.0)


def _mid_body(p, u, dinv, b, w, o):
    h = _gather_h(p, u, dinv, b)
    v = jnp.dot(h, w[...], preferred_element_type=_f32) * dinv[...]
    o[0] = v[:, :DH]
    o[1] = v[:, DH:]


def _fin_body(p, u, dinv, b, wf1, bf1, wf2, bf2, o, acc):
    i = pl.program_id(0)

    @pl.when(i == 0)
    def _():
        acc[...] = jnp.zeros_like(acc)

    h = _gather_h(p, u, dinv, b)
    rows = i * BR + lax.broadcasted_iota(jnp.int32, (BR, 1), 0)
    h = jnp.where(rows < N, h, 0.0)
    acc[0:1, :] += jnp.sum(h, axis=0, keepdims=True)

    @pl.when(i == NB - 1)
    def _():
        g = acc[0:1, :] * (1.0 / N)
        z = jnp.maximum(jnp.dot(g, wf1[...], preferred_element_type=_f32)
                        + bf1[...], 0.0)
        o[...] = jnp.dot(z, wf2[...], preferred_element_type=_f32) + bf2[...]


def _rows_spec():
    return pl.BlockSpec((BR, D), lambda i: (i, 0))


def _half_spec():
    return pl.BlockSpec((2, BR, DH), lambda i: (0, i, 0))


def _pair_spec():
    return pl.BlockSpec((NC, 2, BR, DH), lambda i: (0, 0, i, 0))


def _full_spec(shape):
    return pl.BlockSpec(shape, lambda i: tuple(0 for _ in shape))


_dinv_kernel = pl.pallas_call(
    _dinv_body,
    grid=(NB,),
    in_specs=[pl.BlockSpec((NC, 1, BR, DH), lambda i: (0, 0, i, 0))],
    out_specs=_rows_spec(),
    out_shape=jax.ShapeDtypeStruct((NP, D), _f32),
)

_emb_kernel = pl.pallas_call(
    _emb_body,
    grid=(NB,),
    in_specs=[_rows_spec(), _full_spec((D, H)), _full_spec((1, H)),
              _full_spec((H, H)), _rows_spec()],
    out_specs=_half_spec(),
    out_shape=jax.ShapeDtypeStruct((2, NP, DH), _f32),
)

_mid_kernel = pl.pallas_call(
    _mid_body,
    grid=(NB,),
    in_specs=[_pair_spec(), _half_spec(), _rows_spec(),
              _full_spec((1, H)), _full_spec((H, H))],
    out_specs=_half_spec(),
    out_shape=jax.ShapeDtypeStruct((2, NP, DH), _f32),
)

_fin_kernel = pl.pallas_call(
    _fin_body,
    grid=(NB,),
    in_specs=[_pair_spec(), _half_spec(), _rows_spec(),
              _full_spec((1, H)), _full_spec((H, H)), _full_spec((1, H)),
              _full_spec((H, OUT)), _full_spec((1, OUT))],
    out_specs=pl.BlockSpec((1, OUT), lambda i: (0, 0)),
    out_shape=jax.ShapeDtypeStruct((1, OUT), _f32),
    scratch_shapes=[pltpu.VMEM((8, H), _f32)],
    compiler_params=pltpu.CompilerParams(dimension_semantics=("arbitrary",)),
)


def kernel(x, edge_index, W_emb, b_emb, W1, b1, W2, b2, W3, b3,
           W_fc1, b_fc1, W_fc2, b_fc2):
    ei = edge_index.astype(jnp.int32)
    src3 = jnp.pad(ei[0], (0, EP - E), constant_values=N).reshape(NW, G, C)
    dst3 = jnp.pad(ei[1], (0, EP - E), constant_values=N).reshape(NW, G, C)

    zrows = jnp.zeros((NP, DH), _f32)
    ones1 = jnp.ones((1, NP, DH), _f32)

    degP = _agg1_kernel(ones1, src3, dst3, zrows)
    dinv = _dinv_kernel(degP)

    xp = jnp.pad(x, ((0, NP - N), (0, 0)))

    u = _emb_kernel(xp, W_emb, b_emb.reshape(1, H), W1, dinv)
    for W_next, b_prev in ((W2, b1), (W3, b2)):
        aggP = _agg2_kernel(u, src3, dst3, zrows)
        u = _mid_kernel(aggP, u, dinv, b_prev.reshape(1, H), W_next)

    aggP = _agg2_kernel(u, src3, dst3, zrows)
    out = _fin_kernel(aggP, u, dinv, b3.reshape(1, H),
                      W_fc1, b_fc1.reshape(1, H), W_fc2, b_fc2.reshape(1, OUT))
    return out


# speedup vs baseline: 1.0523x; 1.0523x over previous
import jax
import jax.numpy as jnp
from jax import lax
from jax.experimental import pallas as pl
from jax.experimental.pallas import tpu as pltpu
from jax.experimental.pallas import tpu_sc as plsc

N = 10000
NP = 10240
E = 320000
NC = 2
NS = 16
NW = NC * NS
C = 128
G = 84
EP = NW * G * C
K = 1
NBLK = G // K
RPT = NP // NS
D = 128
DH = 64
H = 128
OUT = 64
BR = 512
NB = NP // BR

_f32 = jnp.float32


def _make_agg_body(nh):
    def body(u, src3, dst3, zrows, out, sidx, didx, gbuf, acc, us, *sems):
        semg = sems[:3]
        semw = sems[3:]
        c = lax.axis_index("c")
        s = lax.axis_index("s")
        wid = c * NS + s
        rsl = pl.ds(s * RPT, RPT)
        pltpu.sync_copy(src3.at[wid], sidx)
        pltpu.sync_copy(dst3.at[wid], didx)

        for h in range(nh):
            pltpu.sync_copy(u.at[h].at[rsl], us.at[rsl])
            uh = us

            def gfire(blk, set_i):
                for j in range(K):
                    pltpu.async_copy(uh.at[sidx.at[K * blk + j]],
                                     gbuf.at[K * set_i + j], semg[set_i])

            def gdrain(set_i):
                for j in range(K):
                    pltpu.make_async_copy(uh.at[sidx.at[0]],
                                          gbuf.at[K * set_i + j],
                                          semg[set_i]).wait()

            def sfire(blk, set_i):
                for j in range(K):
                    pltpu.async_copy(gbuf.at[K * set_i + j],
                                     acc.at[didx.at[K * blk + j]],
                                     semw[set_i], add=True)

            def sdrain(set_i):
                for j in range(K):
                    pltpu.make_async_copy(gbuf.at[K * set_i + j],
                                          acc.at[didx.at[0]],
                                          semw[set_i]).wait()

            pltpu.sync_copy(zrows.at[rsl], acc.at[rsl])
            plsc.subcore_barrier()

            gfire(0, 0)

            def loop(i, carry):
                for pos in range(3):
                    set_nxt = (pos + 1) % 3
                    blk = 3 * i + pos

                    if pos == 2:
                        sdrain(set_nxt)
                    else:
                        @pl.when(i >= 1)
                        def _():
                            sdrain(set_nxt)

                    if pos == 2:
                        @pl.when(i < NBLK // 3 - 1)
                        def _():
                            gfire(blk + 1, set_nxt)
                    else:
                        gfire(blk + 1, set_nxt)

                    gdrain(pos)
                    sfire(blk, pos)
                return carry

            lax.fori_loop(0, NBLK // 3, loop, 0)
            sdrain(1)
            sdrain(2)
            plsc.subcore_barrier()
            pltpu.sync_copy(acc.at[rsl], out.at[c].at[h].at[rsl])
            if h + 1 < nh:
                plsc.subcore_barrier()
    return body


def _deg_body(dst3, zrows, ones_hbm, out, didx, obuf, acc, sem):
    c = lax.axis_index("c")
    s = lax.axis_index("s")
    wid = c * NS + s
    rsl = pl.ds(s * RPT, RPT)
    pltpu.sync_copy(dst3.at[wid], didx)
    pltpu.sync_copy(ones_hbm, obuf)
    pltpu.sync_copy(zrows.at[rsl], acc.at[rsl])
    plsc.subcore_barrier()

    def sfire(g):
        pltpu.async_copy(obuf, acc.at[didx.at[g]], sem, add=True)

    def sdrain():
        pltpu.make_async_copy(obuf, acc.at[didx.at[0]], sem).wait()

    for j in range(3):
        sfire(j)

    def loop(g, carry):
        sdrain()
        sfire(g + 3)
        return carry

    lax.fori_loop(0, G - 3, loop, 0)
    for j in range(3):
        sdrain()
    plsc.subcore_barrier()
    pltpu.sync_copy(acc.at[rsl], out.at[c].at[0].at[rsl])


_sc_mesh = plsc.VectorSubcoreMesh(core_axis_name="c", subcore_axis_name="s")

_deg_kernel = pl.kernel(
    _deg_body,
    out_type=jax.ShapeDtypeStruct((NC, 1, NP, DH), _f32),
    mesh=_sc_mesh,
    scratch_types=[
        pltpu.VMEM((G, C), jnp.int32),
        pltpu.VMEM((C, DH), _f32),
        pltpu.VMEM_SHARED((NP, DH), _f32),
        pltpu.SemaphoreType.DMA,
    ],
    compiler_params=pltpu.CompilerParams(use_tc_tiling_on_sc=False),
)


def _make_agg_kernel(nh):
    return pl.kernel(
        _make_agg_body(nh),
        out_type=jax.ShapeDtypeStruct((NC, nh, NP, DH), _f32),
        mesh=_sc_mesh,
        scratch_types=[
            pltpu.VMEM((G, C), jnp.int32),
            pltpu.VMEM((G, C), jnp.int32),
            pltpu.VMEM((3 * K, C, DH), _f32),
            pltpu.VMEM_SHARED((NP, DH), _f32),
            pltpu.VMEM_SHARED((NP, DH), _f32),
        ] + [pltpu.SemaphoreType.DMA] * 6,
        compiler_params=pltpu.CompilerParams(use_tc_tiling_on_sc=False),
    )


_agg2_kernel = _make_agg_kernel(2)


def _h0_body(x, we, be, o):
    o[...] = jnp.dot(x[...], we[...], preferred_element_type=_f32) + be[...]


def _u1_body(h0, dg, w1, o, od):
    i = pl.program_id(0)
    deg = dg[0, 0][:, 0:1] + dg[1, 0][:, 0:1] + 1.0
    r = lax.rsqrt(deg)
    rows = i * BR + lax.broadcasted_iota(jnp.int32, (BR, 1), 0)
    r = jnp.where(rows < N, r, 0.0)
    dinv = jnp.broadcast_to(r, (BR, D))
    od[...] = dinv
    u = jnp.dot(h0[...], w1[...], preferred_element_type=_f32) * dinv
    o[0] = u[:, :DH]
    o[1] = u[:, DH:]


def _gather_h(p, u, dinv, b):
    pL = p[0, 0] + p[1, 0] + u[0]
    pR = p[0, 1] + p[1, 1] + u[1]
    agg = jnp.concatenate([pL, pR], axis=1)
    return jnp.maximum(agg * dinv[...] + b[...], 0.0)


def _mid_body(p, u, dinv, b, w, o):
    h = _gather_h(p, u, dinv, b)
    v = jnp.dot(h, w[...], preferred_element_type=_f32) * dinv[...]
    o[0] = v[:, :DH]
    o[1] = v[:, DH:]


def _fin_body(p, u, dinv, b, wf1, bf1, wf2, bf2, o, acc):
    i = pl.program_id(0)

    @pl.when(i == 0)
    def _():
        acc[...] = jnp.zeros_like(acc)

    h = _gather_h(p, u, dinv, b)
    rows = i * BR + lax.broadcasted_iota(jnp.int32, (BR, 1), 0)
    h = jnp.where(rows < N, h, 0.0)
    acc[0:1, :] += jnp.sum(h, axis=0, keepdims=True)

    @pl.when(i == NB - 1)
    def _():
        g = acc[0:1, :] * (1.0 / N)
        z = jnp.maximum(jnp.dot(g, wf1[...], preferred_element_type=_f32)
                        + bf1[...], 0.0)
        o[...] = jnp.dot(z, wf2[...], preferred_element_type=_f32) + bf2[...]


def _rows_spec():
    return pl.BlockSpec((BR, D), lambda i: (i, 0))


def _half_spec():
    return pl.BlockSpec((2, BR, DH), lambda i: (0, i, 0))


def _pair_spec():
    return pl.BlockSpec((NC, 2, BR, DH), lambda i: (0, 0, i, 0))


def _full_spec(shape):
    return pl.BlockSpec(shape, lambda i: tuple(0 for _ in shape))


_h0_kernel = pl.pallas_call(
    _h0_body,
    grid=(NB,),
    in_specs=[_rows_spec(), _full_spec((D, H)), _full_spec((1, H))],
    out_specs=_rows_spec(),
    out_shape=jax.ShapeDtypeStruct((NP, H), _f32),
)

_u1_kernel = pl.pallas_call(
    _u1_body,
    grid=(NB,),
    in_specs=[_rows_spec(),
              pl.BlockSpec((NC, 1, BR, DH), lambda i: (0, 0, i, 0)),
              _full_spec((H, H))],
    out_specs=[_half_spec(), _rows_spec()],
    out_shape=[jax.ShapeDtypeStruct((2, NP, DH), _f32),
               jax.ShapeDtypeStruct((NP, D), _f32)],
)

_mid_kernel = pl.pallas_call(
    _mid_body,
    grid=(NB,),
    in_specs=[_pair_spec(), _half_spec(), _rows_spec(),
              _full_spec((1, H)), _full_spec((H, H))],
    out_specs=_half_spec(),
    out_shape=jax.ShapeDtypeStruct((2, NP, DH), _f32),
)

_fin_kernel = pl.pallas_call(
    _fin_body,
    grid=(NB,),
    in_specs=[_pair_spec(), _half_spec(), _rows_spec(),
              _full_spec((1, H)), _full_spec((H, H)), _full_spec((1, H)),
              _full_spec((H, OUT)), _full_spec((1, OUT))],
    out_specs=pl.BlockSpec((1, OUT), lambda i: (0, 0)),
    out_shape=jax.ShapeDtypeStruct((1, OUT), _f32),
    scratch_shapes=[pltpu.VMEM((8, H), _f32)],
    compiler_params=pltpu.CompilerParams(dimension_semantics=("arbitrary",)),
)


def kernel(x, edge_index, W_emb, b_emb, W1, b1, W2, b2, W3, b3,
           W_fc1, b_fc1, W_fc2, b_fc2):
    ei = edge_index.astype(jnp.int32)
    src3 = jnp.pad(ei[0], (0, EP - E), constant_values=N).reshape(NW, G, C)
    dst3 = jnp.pad(ei[1], (0, EP - E), constant_values=N).reshape(NW, G, C)

    zrows = jnp.zeros((NP, DH), _f32)
    ones_c = jnp.ones((C, DH), _f32)

    degP = _deg_kernel(dst3, zrows, ones_c)

    xp = jnp.pad(x, ((0, NP - N), (0, 0)))
    h0 = _h0_kernel(xp, W_emb, b_emb.reshape(1, H))

    u, dinv = _u1_kernel(h0, degP, W1)
    for W_next, b_prev in ((W2, b1), (W3, b2)):
        aggP = _agg2_kernel(u, src3, dst3, zrows)
        u = _mid_kernel(aggP, u, dinv, b_prev.reshape(1, H), W_next)

    aggP = _agg2_kernel(u, src3, dst3, zrows)
    out = _fin_kernel(aggP, u, dinv, b3.reshape(1, H),
                      W_fc1, b_fc1.reshape(1, H), W_fc2, b_fc2.reshape(1, OUT))
    return out


# speedup vs baseline: 1.0538x; 1.0014x over previous
import jax
import jax.numpy as jnp
from jax import lax
from jax.experimental import pallas as pl
from jax.experimental.pallas import tpu as pltpu
from jax.experimental.pallas import tpu_sc as plsc

N = 10000
NP = 10240
E = 320000
NC = 2
NS = 16
NW = NC * NS
C = 96
G = 112
EP = NW * G * C
NSET = 4
NBLK = G
RPT = NP // NS
D = 128
DH = 64
H = 128
OUT = 64
BR = 512
NB = NP // BR

_f32 = jnp.float32


def _make_agg_body(nh):
    def body(u, src3, dst3, zrows, out, sidx, didx, gbuf, acc, us, *sems):
        semg = sems[:NSET]
        semw = sems[NSET:]
        c = lax.axis_index("c")
        s = lax.axis_index("s")
        wid = c * NS + s
        rsl = pl.ds(s * RPT, RPT)
        pltpu.sync_copy(src3.at[wid], sidx)
        pltpu.sync_copy(dst3.at[wid], didx)

        for h in range(nh):
            pltpu.sync_copy(u.at[h].at[rsl], us.at[rsl])
            uh = us

            def gfire(blk, set_i):
                pltpu.async_copy(uh.at[sidx.at[blk]], gbuf.at[set_i],
                                 semg[set_i])

            def gdrain(set_i):
                pltpu.make_async_copy(uh.at[sidx.at[0]], gbuf.at[set_i],
                                      semg[set_i]).wait()

            def sfire(blk, set_i):
                pltpu.async_copy(gbuf.at[set_i], acc.at[didx.at[blk]],
                                 semw[set_i], add=True)

            def sdrain(set_i):
                pltpu.make_async_copy(gbuf.at[set_i], acc.at[didx.at[0]],
                                      semw[set_i]).wait()

            pltpu.sync_copy(zrows.at[rsl], acc.at[rsl])
            plsc.subcore_barrier()

            gfire(0, 0)
            gfire(1, 1)

            def loop(i, carry):
                for pos in range(4):
                    set_pf = (pos + 2) % 4
                    blk = 4 * i + pos

                    if pos >= 2:
                        sdrain(set_pf)
                    else:
                        @pl.when(i >= 1)
                        def _():
                            sdrain(set_pf)

                    if pos < 2:
                        gfire(blk + 2, set_pf)
                    else:
                        @pl.when(i < NBLK // 4 - 1)
                        def _():
                            gfire(blk + 2, set_pf)

                    gdrain(pos)
                    sfire(blk, pos)
                return carry

            lax.fori_loop(0, NBLK // 4, loop, 0)
            sdrain(2)
            sdrain(3)
            plsc.subcore_barrier()
            pltpu.sync_copy(acc.at[rsl], out.at[c].at[h].at[rsl])
            if h + 1 < nh:
                plsc.subcore_barrier()
    return body


def _deg_body(dst3, zrows, ones_hbm, out, didx, obuf, acc, sem):
    c = lax.axis_index("c")
    s = lax.axis_index("s")
    wid = c * NS + s
    rsl = pl.ds(s * RPT, RPT)
    pltpu.sync_copy(dst3.at[wid], didx)
    pltpu.sync_copy(ones_hbm, obuf)
    pltpu.sync_copy(zrows.at[rsl], acc.at[rsl])
    plsc.subcore_barrier()

    def sfire(g):
        pltpu.async_copy(obuf, acc.at[didx.at[g]], sem, add=True)

    def sdrain():
        pltpu.make_async_copy(obuf, acc.at[didx.at[0]], sem).wait()

    for j in range(3):
        sfire(j)

    def loop(g, carry):
        sdrain()
        sfire(g + 3)
        return carry

    lax.fori_loop(0, G - 3, loop, 0)
    for j in range(3):
        sdrain()
    plsc.subcore_barrier()
    pltpu.sync_copy(acc.at[rsl], out.at[c].at[0].at[rsl])


_sc_mesh = plsc.VectorSubcoreMesh(core_axis_name="c", subcore_axis_name="s")

_deg_kernel = pl.kernel(
    _deg_body,
    out_type=jax.ShapeDtypeStruct((NC, 1, NP, DH), _f32),
    mesh=_sc_mesh,
    scratch_types=[
        pltpu.VMEM((G, C), jnp.int32),
        pltpu.VMEM((C, DH), _f32),
        pltpu.VMEM_SHARED((NP, DH), _f32),
        pltpu.SemaphoreType.DMA,
    ],
    compiler_params=pltpu.CompilerParams(use_tc_tiling_on_sc=False),
)


def _make_agg_kernel(nh):
    return pl.kernel(
        _make_agg_body(nh),
        out_type=jax.ShapeDtypeStruct((NC, nh, NP, DH), _f32),
        mesh=_sc_mesh,
        scratch_types=[
            pltpu.VMEM((G, C), jnp.int32),
            pltpu.VMEM((G, C), jnp.int32),
            pltpu.VMEM((NSET, C, DH), _f32),
            pltpu.VMEM_SHARED((NP, DH), _f32),
            pltpu.VMEM_SHARED((NP, DH), _f32),
        ] + [pltpu.SemaphoreType.DMA] * (2 * NSET),
        compiler_params=pltpu.CompilerParams(use_tc_tiling_on_sc=False),
    )


_agg2_kernel = _make_agg_kernel(2)


def _h0_body(x, we, be, o):
    o[...] = jnp.dot(x[...], we[...], preferred_element_type=_f32) + be[...]


def _u1_body(h0, dg, w1, o, od):
    i = pl.program_id(0)
    deg = dg[0, 0][:, 0:1] + dg[1, 0][:, 0:1] + 1.0
    r = lax.rsqrt(deg)
    rows = i * BR + lax.broadcasted_iota(jnp.int32, (BR, 1), 0)
    r = jnp.where(rows < N, r, 0.0)
    dinv = jnp.broadcast_to(r, (BR, D))
    od[...] = dinv
    u = jnp.dot(h0[...], w1[...], preferred_element_type=_f32) * dinv
    o[0] = u[:, :DH]
    o[1] = u[:, DH:]


def _gather_h(p, u, dinv, b):
    pL = p[0, 0] + p[1, 0] + u[0]
    pR = p[0, 1] + p[1, 1] + u[1]
    agg = jnp.concatenate([pL, pR], axis=1)
    return jnp.maximum(agg * dinv[...] + b[...], 0.0)


def _mid_body(p, u, dinv, b, w, o):
    h = _gather_h(p, u, dinv, b)
    v = jnp.dot(h, w[...], preferred_element_type=_f32) * dinv[...]
    o[0] = v[:, :DH]
    o[1] = v[:, DH:]


def _fin_body(p, u, dinv, b, wf1, bf1, wf2, bf2, o, acc):
    i = pl.program_id(0)

    @pl.when(i == 0)
    def _():
        acc[...] = jnp.zeros_like(acc)

    h = _gather_h(p, u, dinv, b)
    rows = i * BR + lax.broadcasted_iota(jnp.int32, (BR, 1), 0)
    h = jnp.where(rows < N, h, 0.0)
    acc[0:1, :] += jnp.sum(h, axis=0, keepdims=True)

    @pl.when(i == NB - 1)
    def _():
        g = acc[0:1, :] * (1.0 / N)
        z = jnp.maximum(jnp.dot(g, wf1[...], preferred_element_type=_f32)
                        + bf1[...], 0.0)
        o[...] = jnp.dot(z, wf2[...], preferred_element_type=_f32) + bf2[...]


def _rows_spec():
    return pl.BlockSpec((BR, D), lambda i: (i, 0))


def _half_spec():
    return pl.BlockSpec((2, BR, DH), lambda i: (0, i, 0))


def _pair_spec():
    return pl.BlockSpec((NC, 2, BR, DH), lambda i: (0, 0, i, 0))


def _full_spec(shape):
    return pl.BlockSpec(shape, lambda i: tuple(0 for _ in shape))


_h0_kernel = pl.pallas_call(
    _h0_body,
    grid=(NB,),
    in_specs=[_rows_spec(), _full_spec((D, H)), _full_spec((1, H))],
    out_specs=_rows_spec(),
    out_shape=jax.ShapeDtypeStruct((NP, H), _f32),
)

_u1_kernel = pl.pallas_call(
    _u1_body,
    grid=(NB,),
    in_specs=[_rows_spec(),
              pl.BlockSpec((NC, 1, BR, DH), lambda i: (0, 0, i, 0)),
              _full_spec((H, H))],
    out_specs=[_half_spec(), _rows_spec()],
    out_shape=[jax.ShapeDtypeStruct((2, NP, DH), _f32),
               jax.ShapeDtypeStruct((NP, D), _f32)],
)

_mid_kernel = pl.pallas_call(
    _mid_body,
    grid=(NB,),
    in_specs=[_pair_spec(), _half_spec(), _rows_spec(),
              _full_spec((1, H)), _full_spec((H, H))],
    out_specs=_half_spec(),
    out_shape=jax.ShapeDtypeStruct((2, NP, DH), _f32),
)

_fin_kernel = pl.pallas_call(
    _fin_body,
    grid=(NB,),
    in_specs=[_pair_spec(), _half_spec(), _rows_spec(),
              _full_spec((1, H)), _full_spec((H, H)), _full_spec((1, H)),
              _full_spec((H, OUT)), _full_spec((1, OUT))],
    out_specs=pl.BlockSpec((1, OUT), lambda i: (0, 0)),
    out_shape=jax.ShapeDtypeStruct((1, OUT), _f32),
    scratch_shapes=[pltpu.VMEM((8, H), _f32)],
    compiler_params=pltpu.CompilerParams(dimension_semantics=("arbitrary",)),
)


def kernel(x, edge_index, W_emb, b_emb, W1, b1, W2, b2, W3, b3,
           W_fc1, b_fc1, W_fc2, b_fc2):
    ei = edge_index.astype(jnp.int32)
    src3 = jnp.pad(ei[0], (0, EP - E), constant_values=N).reshape(NW, G, C)
    dst3 = jnp.pad(ei[1], (0, EP - E), constant_values=N).reshape(NW, G, C)

    zrows = jnp.zeros((NP, DH), _f32)
    ones_c = jnp.ones((C, DH), _f32)

    degP = _deg_kernel(dst3, zrows, ones_c)

    xp = jnp.pad(x, ((0, NP - N), (0, 0)))
    h0 = _h0_kernel(xp, W_emb, b_emb.reshape(1, H))

    u, dinv = _u1_kernel(h0, degP, W1)
    for W_next, b_prev in ((W2, b1), (W3, b2)):
        aggP = _agg2_kernel(u, src3, dst3, zrows)
        u = _mid_kernel(aggP, u, dinv, b_prev.reshape(1, H), W_next)

    aggP = _agg2_kernel(u, src3, dst3, zrows)
    out = _fin_kernel(aggP, u, dinv, b3.reshape(1, H),
                      W_fc1, b_fc1.reshape(1, H), W_fc2, b_fc2.reshape(1, OUT))
    return out
